# baseline (speedup 1.0000x reference)
import functools

import jax
import jax.numpy as jnp
from jax import lax
from jax.experimental import pallas as pl
from jax.experimental.pallas import tpu as pltpu
from jax.experimental.pallas import tpu_sc as plsc

V = 100000
D = 64
B = 1024
CTX = 20

NC = 2
NS = 16
NW = NC * NS
B_PER_W = B // NW
ROWS_PER_W = B_PER_W * CTX
IDX_CHUNK = 128
NCHUNK = ROWS_PER_W // IDX_CHUNK

_SC_MESH = plsc.VectorSubcoreMesh(
    core_axis_name="c", subcore_axis_name="s", num_cores=NC, num_subcores=NS
)


@functools.partial(
    pl.kernel,
    out_type=jax.ShapeDtypeStruct((B, D), jnp.float32),
    mesh=_SC_MESH,
    scratch_types=[
        pltpu.VMEM((NCHUNK, IDX_CHUNK), jnp.int32),
        pltpu.VMEM((ROWS_PER_W, D), jnp.float32),
        pltpu.VMEM((B_PER_W, D), jnp.float32),
        pltpu.SemaphoreType.DMA,
    ],
    compiler_params=pltpu.CompilerParams(use_tc_tiling_on_sc=False),
)
def _sc_pool(ctx_hbm, table_hbm, out_hbm, idx_v, rows_v, pooled_v, sem):
    wid = lax.axis_index("s") * NC + lax.axis_index("c")
    pltpu.sync_copy(ctx_hbm.at[wid], idx_v)
    copies = [
        pltpu.async_copy(
            table_hbm.at[idx_v.at[j]],
            rows_v.at[pl.ds(j * IDX_CHUNK, IDX_CHUNK)],
            sem,
        )
        for j in range(NCHUNK)
    ]
    for c in copies:
        c.wait()

    def body(i, carry):
        base = i * CTX
        for d in range(D // 16):
            acc = rows_v[base, pl.ds(d * 16, 16)]
            for c in range(1, CTX):
                acc = acc + rows_v[base + c, pl.ds(d * 16, 16)]
            pooled_v[i, pl.ds(d * 16, 16)] = acc * (1.0 / CTX)
        return carry

    lax.fori_loop(0, B_PER_W, body, 0)
    pltpu.sync_copy(pooled_v, out_hbm.at[pl.ds(wid * B_PER_W, B_PER_W)])


VBLK = 2048
NVB = pl.cdiv(V, VBLK)


def _mm_body(p_ref, w_ref, b_ref, o_ref):
    o_ref[...] = (
        lax.dot_general(
            p_ref[...],
            w_ref[...],
            (((1,), (1,)), ((), ())),
            preferred_element_type=jnp.float32,
        )
        + b_ref[...]
    )


_matmul = pl.pallas_call(
    _mm_body,
    grid=(NVB,),
    in_specs=[
        pl.BlockSpec((B, D), lambda v: (0, 0)),
        pl.BlockSpec((VBLK, D), lambda v: (v, 0)),
        pl.BlockSpec((1, VBLK), lambda v: (0, v)),
    ],
    out_specs=pl.BlockSpec((B, VBLK), lambda v: (0, v)),
    out_shape=jax.ShapeDtypeStruct((B, V), jnp.float32),
    compiler_params=pltpu.CompilerParams(dimension_semantics=("parallel",)),
)


def kernel(context, emb_table, W, b):
    ctx = context.astype(jnp.int32).reshape(NW, NCHUNK, IDX_CHUNK)
    pooled = _sc_pool(ctx, emb_table)
    return _matmul(pooled, W, b.reshape(1, V))


# speedup vs baseline: 2.7288x; 2.7288x over previous
import functools

import jax
import jax.numpy as jnp
from jax import lax
from jax.experimental import pallas as pl
from jax.experimental.pallas import tpu as pltpu
from jax.experimental.pallas import tpu_sc as plsc

V = 100000
D = 64
B = 1024
CTX = 20

NC = 2
NS = 16
NW = NC * NS
B_PER_W = B // NW
ROWS_PER_W = B_PER_W * CTX
IDX_CHUNK = 128
NCHUNK = ROWS_PER_W // IDX_CHUNK

_SC_MESH = plsc.VectorSubcoreMesh(
    core_axis_name="c", subcore_axis_name="s", num_cores=NC, num_subcores=NS
)


@functools.partial(
    pl.kernel,
    out_type=jax.ShapeDtypeStruct((B, D), jnp.float32),
    mesh=_SC_MESH,
    scratch_types=[
        pltpu.VMEM((NCHUNK, IDX_CHUNK), jnp.int32),
        pltpu.VMEM((ROWS_PER_W, D), jnp.float32),
        pltpu.VMEM((B_PER_W, D), jnp.float32),
        pltpu.SemaphoreType.DMA,
    ],
    compiler_params=pltpu.CompilerParams(use_tc_tiling_on_sc=False),
)
def _sc_pool(ctx_hbm, table_hbm, out_hbm, idx_v, rows_v, pooled_v, sem):
    wid = lax.axis_index("s") * NC + lax.axis_index("c")
    pltpu.sync_copy(ctx_hbm.at[wid], idx_v)
    copies = [
        pltpu.async_copy(
            table_hbm.at[idx_v.at[j]],
            rows_v.at[pl.ds(j * IDX_CHUNK, IDX_CHUNK)],
            sem,
        )
        for j in range(NCHUNK)
    ]
    for c in copies:
        c.wait()

    def body(i, carry):
        base = i * CTX
        for d in range(D // 16):
            acc = rows_v[base, pl.ds(d * 16, 16)]
            for c in range(1, CTX):
                acc = acc + rows_v[base + c, pl.ds(d * 16, 16)]
            pooled_v[i, pl.ds(d * 16, 16)] = acc * (1.0 / CTX)
        return carry

    lax.fori_loop(0, B_PER_W, body, 0)
    pltpu.sync_copy(pooled_v, out_hbm.at[pl.ds(wid * B_PER_W, B_PER_W)])


VBLK = 2048
NVB = pl.cdiv(V, VBLK)


def _mm_body(wt_ref, p_ref, b_ref, o_ref):
    o_ref[...] = (
        lax.dot_general(
            wt_ref[...],
            p_ref[...],
            (((0,), (1,)), ((), ())),
            preferred_element_type=jnp.float32,
        )
        + b_ref[...].reshape(VBLK, 1)
    )


_matmul = pl.pallas_call(
    _mm_body,
    grid=(NVB,),
    in_specs=[
        pl.BlockSpec((D, VBLK), lambda v: (0, v)),
        pl.BlockSpec((B, D), lambda v: (0, 0)),
        pl.BlockSpec((VBLK,), lambda v: (v,)),
    ],
    out_specs=pl.BlockSpec((VBLK, B), lambda v: (v, 0)),
    out_shape=jax.ShapeDtypeStruct((V, B), jnp.float32),
    compiler_params=pltpu.CompilerParams(
        dimension_semantics=("parallel",),
        vmem_limit_bytes=100 * 1024 * 1024,
    ),
)


def kernel(context, emb_table, W, b):
    ctx = context.astype(jnp.int32).reshape(NW, NCHUNK, IDX_CHUNK)
    pooled = _sc_pool(ctx, emb_table)
    out_t = _matmul(W.T, pooled, b)
    return out_t.T


# speedup vs baseline: 2.9111x; 1.0668x over previous
import functools

import jax
import jax.numpy as jnp
from jax import lax
from jax.experimental import pallas as pl
from jax.experimental.pallas import tpu as pltpu
from jax.experimental.pallas import tpu_sc as plsc

V = 100000
D = 64
B = 1024
CTX = 20

NC = 2
NS = 16
NW = NC * NS
D_PER_W = D // NW
NLANE = 16
NGRP = B // NLANE

_SC_MESH = plsc.VectorSubcoreMesh(
    core_axis_name="c", subcore_axis_name="s", num_cores=NC, num_subcores=NS
)


@functools.partial(
    pl.kernel,
    out_type=jax.ShapeDtypeStruct((D, NGRP, NLANE), jnp.float32),
    mesh=_SC_MESH,
    scratch_types=[
        pltpu.VMEM((CTX, NGRP, NLANE), jnp.int32),
        pltpu.VMEM((V,), jnp.float32),
        pltpu.VMEM((D_PER_W, NGRP, NLANE), jnp.float32),
    ],
    compiler_params=pltpu.CompilerParams(
        needs_layout_passes=False, use_tc_tiling_on_sc=False
    ),
)
def _sc_pool(ctx_hbm, table_t_hbm, out_hbm, ctx_v, row_v, pooled_v):
    wid = lax.axis_index("s") * NC + lax.axis_index("c")
    pltpu.sync_copy(ctx_hbm, ctx_v)
    for di in range(D_PER_W):
        d = wid * D_PER_W + di
        pltpu.sync_copy(table_t_hbm.at[d], row_v)

        def group(g, carry):
            acc = plsc.load_gather(row_v, [ctx_v[0, g, :]])
            for c in range(1, CTX):
                acc = acc + plsc.load_gather(row_v, [ctx_v[c, g, :]])
            pooled_v[di, g, :] = acc * (1.0 / CTX)
            return carry

        lax.fori_loop(0, NGRP, group, 0)
    pltpu.sync_copy(pooled_v, out_hbm.at[pl.ds(wid * D_PER_W, D_PER_W)])


VBLK = 2048
NVB = pl.cdiv(V, VBLK)


def _mm_body(wt_ref, p_ref, b_ref, o_ref):
    o_ref[...] = (
        lax.dot_general(
            wt_ref[...],
            p_ref[...],
            (((0,), (0,)), ((), ())),
            preferred_element_type=jnp.float32,
        )
        + b_ref[...].reshape(VBLK, 1)
    )


_matmul = pl.pallas_call(
    _mm_body,
    grid=(NVB,),
    in_specs=[
        pl.BlockSpec((D, VBLK), lambda v: (0, v)),
        pl.BlockSpec((D, B), lambda v: (0, 0)),
        pl.BlockSpec((VBLK,), lambda v: (v,)),
    ],
    out_specs=pl.BlockSpec((VBLK, B), lambda v: (v, 0)),
    out_shape=jax.ShapeDtypeStruct((V, B), jnp.float32),
    compiler_params=pltpu.CompilerParams(
        dimension_semantics=("parallel",),
        vmem_limit_bytes=100 * 1024 * 1024,
    ),
)


def kernel(context, emb_table, W, b):
    ctx_t = context.T.astype(jnp.int32).reshape(CTX, NGRP, NLANE)
    pooled_t = _sc_pool(ctx_t, emb_table.T).reshape(D, B)
    out_t = _matmul(W.T, pooled_t, b)
    return out_t.T
